# baseline (speedup 1.0000x reference)
import jax
import jax.numpy as jnp
from jax.experimental import pallas as pl
from jax.experimental.pallas import tpu as pltpu

C, P, F, B = 100, 64, 128, 4096
ALPHA = 5.0
EPS = 1e-08

TB = 2048
NBT = B // TB
CCHUNK = 20
RCHUNK = CCHUNK * P
NCHUNK = C // CCHUNK


def _cce_kernel(x_ref, tgt_ref, y_ref, sum_ref, acc_ref):
    i = pl.program_id(0)
    j = pl.program_id(1)

    @pl.when((i == 0) & (j == 0))
    def _init_out():
        sum_ref[...] = jnp.zeros_like(sum_ref)

    @pl.when(j == 0)
    def _init_acc():
        acc_ref[...] = jnp.full_like(acc_ref, jnp.inf)

    x = x_ref[...]
    y = y_ref[...]
    ym = -2.0 * y
    y2 = jnp.sum(y * y, axis=1)
    s = y2[:, None] + jax.lax.dot_general(
        ym, x, (((1,), (1,)), ((), ())),
        preferred_element_type=jnp.float32)
    m = jnp.min(s.reshape(CCHUNK, P, TB), axis=1)
    cls = jax.lax.broadcasted_iota(jnp.int32, (CCHUNK, TB), 0) + j * CCHUNK
    tgt = tgt_ref[0, 0, :]
    is_t = cls == tgt[None, :]
    tmin = jnp.min(jnp.where(is_t, m, jnp.inf), axis=0)
    wmin = jnp.min(jnp.where(is_t, jnp.inf, m), axis=0)
    acc_ref[0, :] = jnp.minimum(acc_ref[0, :], tmin)
    acc_ref[1, :] = jnp.minimum(acc_ref[1, :], wmin)

    @pl.when(j == NCHUNK - 1)
    def _finish_tile():
        x2 = jnp.sum(x * x, axis=1)
        t = jnp.maximum(acc_ref[0, :] + x2, 0.0)
        w = jnp.maximum(acc_ref[1, :] + x2, 0.0)
        tp = jnp.sum(t.reshape(TB // 128, 128), axis=0)
        wp = jnp.sum(w.reshape(TB // 128, 128), axis=0)
        sum_ref[...] += jnp.stack([tp, wp])[None]


@jax.jit
def kernel(outputs, target_classes, clusters):
    y = clusters.reshape(C * P, F)
    tgt = target_classes.astype(jnp.int32).reshape(NBT, 1, TB)

    sums = pl.pallas_call(
        _cce_kernel,
        grid=(NBT, NCHUNK),
        in_specs=[
            pl.BlockSpec((TB, F), lambda i, j: (i, 0)),
            pl.BlockSpec((1, 1, TB), lambda i, j: (i, 0, 0)),
            pl.BlockSpec((RCHUNK, F), lambda i, j: (j, 0)),
        ],
        out_specs=pl.BlockSpec((1, 2, 128), lambda i, j: (0, 0, 0)),
        out_shape=jax.ShapeDtypeStruct((1, 2, 128), jnp.float32),
        scratch_shapes=[pltpu.VMEM((2, TB), jnp.float32)],
    )(outputs, tgt, y)

    denom = float(B * F)
    target_loss = jnp.sum(sums[0, 0]) / denom
    non_target_loss = jnp.sum(sums[0, 1]) / denom
    return (1.0 - ALPHA) * target_loss + ALPHA / (non_target_loss + EPS)


# speedup vs baseline: 1.1890x; 1.1890x over previous
import jax
import jax.numpy as jnp
from jax.experimental import pallas as pl

C, P, F, B = 100, 64, 128, 4096
ALPHA = 5.0
EPS = 1e-08

TB = 2048
NBT = B // TB
CCHUNK = 20
RCHUNK = CCHUNK * P


def _cce_kernel(x_ref, tgt_ref, y_ref, sum_ref):
    i = pl.program_id(0)

    x = x_ref[...]
    x2 = jnp.sum(x * x, axis=1)
    xm = -2.0 * x
    tgt = tgt_ref[0, 0, :]

    tmin = jnp.full((TB,), jnp.inf, jnp.float32)
    wmin = jnp.full((TB,), jnp.inf, jnp.float32)

    for j in range(C * P // RCHUNK):
        y = y_ref[j * RCHUNK:(j + 1) * RCHUNK, :]
        y2 = jnp.sum(y * y, axis=1)
        s = y2[:, None] + jax.lax.dot_general(
            y, xm, (((1,), (1,)), ((), ())),
            preferred_element_type=jnp.float32)
        m = jnp.min(s.reshape(CCHUNK, P, TB), axis=1)
        cls = jax.lax.broadcasted_iota(jnp.int32, (CCHUNK, TB), 0) + j * CCHUNK
        is_t = cls == tgt[None, :]
        tmin = jnp.minimum(tmin, jnp.min(jnp.where(is_t, m, jnp.inf), axis=0))
        wmin = jnp.minimum(wmin, jnp.min(jnp.where(is_t, jnp.inf, m), axis=0))

    t = jnp.maximum(tmin + x2, 0.0)
    w = jnp.maximum(wmin + x2, 0.0)
    tp = jnp.sum(t.reshape(TB // 128, 128), axis=0)
    wp = jnp.sum(w.reshape(TB // 128, 128), axis=0)

    @pl.when(i == 0)
    def _init():
        sum_ref[...] = jnp.zeros_like(sum_ref)

    sum_ref[...] += jnp.stack([tp, wp])[None]


@jax.jit
def kernel(outputs, target_classes, clusters):
    y = clusters.reshape(C * P, F)
    tgt = target_classes.astype(jnp.int32).reshape(NBT, 1, TB)

    sums = pl.pallas_call(
        _cce_kernel,
        grid=(NBT,),
        in_specs=[
            pl.BlockSpec((TB, F), lambda i: (i, 0)),
            pl.BlockSpec((1, 1, TB), lambda i: (i, 0, 0)),
            pl.BlockSpec((C * P, F), lambda i: (0, 0)),
        ],
        out_specs=pl.BlockSpec((1, 2, 128), lambda i: (0, 0, 0)),
        out_shape=jax.ShapeDtypeStruct((1, 2, 128), jnp.float32),
    )(outputs, tgt, y)

    denom = float(B * F)
    target_loss = jnp.sum(sums[0, 0]) / denom
    non_target_loss = jnp.sum(sums[0, 1]) / denom
    return (1.0 - ALPHA) * target_loss + ALPHA / (non_target_loss + EPS)


# speedup vs baseline: 1.3768x; 1.1579x over previous
import jax
import jax.numpy as jnp
from jax.experimental import pallas as pl
from jax.experimental.pallas import tpu as pltpu

C, P, F, B = 100, 64, 128, 4096
ALPHA = 5.0
EPS = 1e-08

TB = 2048
NBT = B // TB
CCHUNK = 20
RCHUNK = CCHUNK * P


def _cce_kernel(x_ref, tgt_ref, y_ref, out_ref, acc_ref):
    i = pl.program_id(0)

    x = x_ref[...]
    x2 = jnp.sum(x * x, axis=1)
    xm = -2.0 * x
    tgt = tgt_ref[0, 0, :]

    tmin = jnp.full((TB,), jnp.inf, jnp.float32)
    wmin = jnp.full((TB,), jnp.inf, jnp.float32)

    for j in range(C * P // RCHUNK):
        y = y_ref[j * RCHUNK:(j + 1) * RCHUNK, :]
        y2 = jnp.sum(y * y, axis=1)
        s = y2[:, None] + jax.lax.dot_general(
            y, xm, (((1,), (1,)), ((), ())),
            preferred_element_type=jnp.float32)
        m = jnp.min(s.reshape(CCHUNK, P, TB), axis=1)
        cls = jax.lax.broadcasted_iota(jnp.int32, (CCHUNK, TB), 0) + j * CCHUNK
        is_t = cls == tgt[None, :]
        tmin = jnp.minimum(tmin, jnp.min(jnp.where(is_t, m, jnp.inf), axis=0))
        wmin = jnp.minimum(wmin, jnp.min(jnp.where(is_t, jnp.inf, m), axis=0))

    t = jnp.maximum(tmin + x2, 0.0)
    w = jnp.maximum(wmin + x2, 0.0)
    tp = jnp.sum(t.reshape(TB // 128, 128), axis=0)
    wp = jnp.sum(w.reshape(TB // 128, 128), axis=0)

    @pl.when(i == 0)
    def _init():
        acc_ref[...] = jnp.zeros_like(acc_ref)

    acc_ref[...] += jnp.stack([tp, wp])

    @pl.when(i == NBT - 1)
    def _finish():
        denom = float(B * F)
        target_loss = jnp.sum(acc_ref[0, :]) / denom
        non_target_loss = jnp.sum(acc_ref[1, :]) / denom
        out_ref[0] = (1.0 - ALPHA) * target_loss \
            + ALPHA / (non_target_loss + EPS)


@jax.jit
def kernel(outputs, target_classes, clusters):
    y = clusters.reshape(C * P, F)
    tgt = target_classes.astype(jnp.int32).reshape(NBT, 1, TB)

    loss = pl.pallas_call(
        _cce_kernel,
        grid=(NBT,),
        in_specs=[
            pl.BlockSpec((TB, F), lambda i: (i, 0)),
            pl.BlockSpec((1, 1, TB), lambda i: (i, 0, 0)),
            pl.BlockSpec((C * P, F), lambda i: (0, 0)),
        ],
        out_specs=pl.BlockSpec(memory_space=pltpu.SMEM),
        out_shape=jax.ShapeDtypeStruct((1,), jnp.float32),
        scratch_shapes=[pltpu.VMEM((2, 128), jnp.float32)],
    )(outputs, tgt, y)

    return loss[0]
